# baseline (speedup 1.0000x reference)
import functools

import jax
import jax.numpy as jnp
from jax import lax
from jax.experimental import pallas as pl
from jax.experimental.pallas import tpu as pltpu
from jax.experimental.pallas import tpu_sc as plsc

VOCAB = 1000000
DIM = 32
B = 4096
L = 200

NUM_WORKERS = 32
CHUNK = 128
N_FLAT = B * L
N_CHUNKS = N_FLAT // CHUNK
CHUNKS_PER_WORKER = N_CHUNKS // NUM_WORKERS
BT = B // CHUNK
CD = CHUNK * DIM
K = 5
NT = CHUNKS_PER_WORKER // (2 * K)


def _make_sc_gather():
  mesh = plsc.VectorSubcoreMesh(core_axis_name="c", subcore_axis_name="s")

  @functools.partial(
      pl.kernel,
      mesh=mesh,
      out_type=jax.ShapeDtypeStruct((L, DIM // 8, BT, 8 * CHUNK), jnp.float32),
      compiler_params=pltpu.CompilerParams(
          use_tc_tiling_on_sc=False, needs_layout_passes=False),
      scratch_types=[
          pltpu.VMEM((CHUNKS_PER_WORKER, CHUNK), jnp.int32),
          pltpu.VMEM((2 * K, CHUNK, DIM), jnp.float32),
          pltpu.VMEM((2 * K, CD), jnp.float32),
          pltpu.SemaphoreType.DMA,
          pltpu.SemaphoreType.DMA,
          pltpu.SemaphoreType.DMA,
          pltpu.SemaphoreType.DMA,
      ],
  )
  def gather_kernel(table_hbm, idx_hbm, out_hbm, idx_v, rows_v, ct_v,
                    ga, gb, sa, sb):
    wid = lax.axis_index("s") * 2 + lax.axis_index("c")
    chunk_base = wid * CHUNKS_PER_WORKER
    pltpu.sync_copy(idx_hbm.at[pl.ds(chunk_base, CHUNKS_PER_WORKER)], idx_v)

    iota16 = lax.iota(jnp.int32, 16)
    row_vecs = [iota16 + (i0 * 16) for i0 in range(8)]

    def gath(t, half, s, sem):
      return pltpu.make_async_copy(
          table_hbm.at[idx_v.at[t]], rows_v.at[half * K + s], sem)

    def stores(t, half, s, sem):
      j = chunk_base + t
      lrow = j // BT
      btile = j % BT
      return [
          pltpu.make_async_copy(
              ct_v.at[half * K + s, pl.ds(dt * (8 * CHUNK), 8 * CHUNK)],
              out_hbm.at[lrow, dt, btile], sem)
          for dt in range(4)
      ]

    def transpose_half(half):
      def sbody(s, carry):
        slot = half * K + s
        src = rows_v.at[slot]
        dst = ct_v.at[slot]
        for d in range(DIM):
          col = jnp.full((16,), d, jnp.int32)
          for i0 in range(8):
            v = plsc.load_gather(src, [row_vecs[i0], col])
            dst[pl.ds(d * CHUNK + i0 * 16, 16)] = v
        return carry

      lax.fori_loop(0, K, sbody, 0)

    for s in range(K):
      gath(s, 0, s, ga).start()

    def body(t, carry):
      base = t * 2 * K
      for s in range(K):
        gath(base + s, 0, s, ga).wait()
      @pl.when(t > 0)
      def _():
        for s in range(K):
          for c in stores(base - K + s, 1, s, sb):
            c.wait()
      for s in range(K):
        gath(base + K + s, 1, s, gb).start()
      transpose_half(0)
      for s in range(K):
        for c in stores(base + s, 0, s, sa):
          c.start()
      for s in range(K):
        gath(base + K + s, 1, s, gb).wait()
      transpose_half(1)
      @pl.when(t < NT - 1)
      def _():
        for s in range(K):
          gath(base + 2 * K + s, 0, s, ga).start()
      for s in range(K):
        for c in stores(base + s, 0, s, sa):
          c.wait()
      for s in range(K):
        for c in stores(base + K + s, 1, s, sb):
          c.start()
      return carry

    lax.fori_loop(0, NT, body, 0)
    last = (NT - 1) * 2 * K + K
    for s in range(K):
      for c in stores(last + s, 1, s, sb):
        c.wait()

  return gather_kernel


_sc_gather = _make_sc_gather()


def kernel(inp, table):
  idx = jnp.swapaxes(inp, 0, 1).astype(jnp.int32).reshape(N_CHUNKS, CHUNK)
  out4 = _sc_gather(table, idx)
  out5 = out4.reshape(L, DIM // 8, BT, 8, CHUNK)
  return out5.transpose(2, 4, 0, 1, 3).reshape(B, L, DIM)


# speedup vs baseline: 1.2303x; 1.2303x over previous
import functools

import jax
import jax.numpy as jnp
from jax import lax
from jax.experimental import pallas as pl
from jax.experimental.pallas import tpu as pltpu
from jax.experimental.pallas import tpu_sc as plsc

VOCAB = 1000000
DIM = 32
B = 4096
L = 200

NUM_WORKERS = 32
CHUNK = 128
N_FLAT = B * L
N_CHUNKS = N_FLAT // CHUNK
CHUNKS_PER_WORKER = N_CHUNKS // NUM_WORKERS
BT = B // CHUNK
CD = CHUNK * DIM
K = 5
NT = CHUNKS_PER_WORKER // (2 * K)


def _make_sc_gather():
  mesh = plsc.VectorSubcoreMesh(core_axis_name="c", subcore_axis_name="s")

  @functools.partial(
      pl.kernel,
      mesh=mesh,
      out_type=jax.ShapeDtypeStruct((L, DIM // 8, BT, 8 * CHUNK), jnp.float32),
      compiler_params=pltpu.CompilerParams(
          use_tc_tiling_on_sc=False, needs_layout_passes=False),
      scratch_types=[
          pltpu.VMEM((CHUNKS_PER_WORKER, CHUNK), jnp.int32),
          pltpu.VMEM((2 * K, CHUNK, DIM), jnp.float32),
          pltpu.VMEM((2 * K, CD), jnp.float32),
          pltpu.SemaphoreType.DMA,
          pltpu.SemaphoreType.DMA,
          pltpu.SemaphoreType.DMA,
          pltpu.SemaphoreType.DMA,
      ],
  )
  def gather_kernel(table_hbm, idx_hbm, out_hbm, idx_v, rows_v, ct_v,
                    ga, gb, sa, sb):
    wid = lax.axis_index("s") * 2 + lax.axis_index("c")
    chunk_base = wid * CHUNKS_PER_WORKER
    pltpu.sync_copy(idx_hbm.at[pl.ds(chunk_base, CHUNKS_PER_WORKER)], idx_v)

    iota16 = lax.iota(jnp.int32, 16)
    row_vecs = [iota16 + (i0 * 16) for i0 in range(8)]

    def gath(t, half, s, sem):
      return pltpu.make_async_copy(
          table_hbm.at[idx_v.at[t]], rows_v.at[half * K + s], sem)

    def stores(t, half, s, sem):
      j = chunk_base + t
      lrow = j // BT
      btile = j % BT
      return [
          pltpu.make_async_copy(
              ct_v.at[half * K + s, pl.ds(dt * (8 * CHUNK), 8 * CHUNK)],
              out_hbm.at[lrow, dt, btile], sem)
          for dt in range(4)
      ]

    def transpose_half(half):
      def sbody(s, carry):
        slot = half * K + s
        src = rows_v.at[slot]
        dst = ct_v.at[slot]
        for d0 in range(0, DIM, 2):
          vs = []
          for d in (d0, d0 + 1):
            col = jnp.full((16,), d, jnp.int32)
            for i0 in range(8):
              vs.append(plsc.load_gather(src, [row_vecs[i0], col]))
          for k, v in enumerate(vs):
            d = d0 + k // 8
            i0 = k % 8
            dst[pl.ds(d * CHUNK + i0 * 16, 16)] = v
        return carry

      lax.fori_loop(0, K, sbody, 0)

    for s in range(K):
      gath(s, 0, s, ga).start()

    def body(t, carry):
      base = t * 2 * K
      for s in range(K):
        gath(base + s, 0, s, ga).wait()
      @pl.when(t > 0)
      def _():
        for s in range(K):
          for c in stores(base - K + s, 1, s, sb):
            c.wait()
      for s in range(K):
        gath(base + K + s, 1, s, gb).start()
      transpose_half(0)
      for s in range(K):
        for c in stores(base + s, 0, s, sa):
          c.start()
      for s in range(K):
        gath(base + K + s, 1, s, gb).wait()
      transpose_half(1)
      @pl.when(t < NT - 1)
      def _():
        for s in range(K):
          gath(base + 2 * K + s, 0, s, ga).start()
      for s in range(K):
        for c in stores(base + s, 0, s, sa):
          c.wait()
      for s in range(K):
        for c in stores(base + K + s, 1, s, sb):
          c.start()
      return carry

    lax.fori_loop(0, NT, body, 0)
    last = (NT - 1) * 2 * K + K
    for s in range(K):
      for c in stores(last + s, 1, s, sb):
        c.wait()

  return gather_kernel


_sc_gather = _make_sc_gather()


def kernel(inp, table):
  idx = jnp.swapaxes(inp, 0, 1).astype(jnp.int32).reshape(N_CHUNKS, CHUNK)
  out4 = _sc_gather(table, idx)
  out5 = out4.reshape(L, DIM // 8, BT, 8, CHUNK)
  return out5.transpose(2, 4, 0, 1, 3).reshape(B, L, DIM)
